# baseline (speedup 1.0000x reference)
import jax
import jax.numpy as jnp
from jax.experimental import pallas as pl
from jax.experimental.pallas import tpu as pltpu

B, S, E, H = 64, 32, 1024, 1024
G4 = 4 * H
SPLIT = 50

_DN = (((1,), (0,)), ((), ()))
_DNT = (((1,), (1,)), ((), ()))


def _proj_kernel(x_ref, w0_ref, w1_ref, b0_ref, b1_ref, m_ref, out_ref):
    x = x_ref[...]
    z0 = jax.lax.dot_general(x, w0_ref[...], _DNT,
                             preferred_element_type=jnp.float32)
    z1 = jax.lax.dot_general(x, w1_ref[...], _DNT,
                             preferred_element_type=jnp.float32)
    m = m_ref[...]
    out_ref[...] = (m * (z0 + b0_ref[...])
                    + (1.0 - m) * (z1 + b1_ref[...])).astype(jnp.bfloat16)


def _step_kernel(z_hbm, m_ref, wh0_ref, wh1_ref, out_hbm, hT_ref, cT_ref,
                 wbf_ref, h_ref, c_ref, ga_ref, gb_ref, zb0, zb1, ob0, ob1,
                 zs0, zs1, os0, os1):
    wbf_ref[:H, :] = jnp.swapaxes(wh0_ref[...], 0, 1).astype(jnp.bfloat16)
    wbf_ref[H:, :] = jnp.swapaxes(wh1_ref[...], 0, 1).astype(jnp.bfloat16)
    h_ref[...] = jnp.zeros_like(h_ref)
    c_ref[...] = jnp.zeros_like(c_ref)
    ga_ref[...] = jnp.zeros_like(ga_ref)
    pltpu.make_async_copy(z_hbm.at[0], zb0, zs0).start()
    pltpu.make_async_copy(z_hbm.at[1], zb1, zs1).start()

    def one_step(t, zbuf, zsem, obuf, osem, gcur, gnext):
        pltpu.make_async_copy(z_hbm.at[t], zbuf, zsem).wait()
        gates = gcur[...] + zbuf[...]
        i = jax.nn.sigmoid(gates[:, :H])
        f = jax.nn.sigmoid(gates[:, H:2 * H])
        gg = jnp.tanh(gates[:, 2 * H:3 * H])
        o = jax.nn.sigmoid(gates[:, 3 * H:])
        c = f * c_ref[...] + i * gg
        h2 = o * jnp.tanh(c)
        c_ref[...] = c
        h_ref[...] = h2
        m2 = m_ref[t + 1]
        hcat = jnp.concatenate([m2 * h2, h2 - m2 * h2],
                               axis=1).astype(jnp.bfloat16)
        gnext[...] = jax.lax.dot_general(hcat, wbf_ref[...], _DN,
                                         preferred_element_type=jnp.float32)

        @pl.when(t >= 2)
        def _():
            pltpu.make_async_copy(obuf, out_hbm.at[t], osem).wait()

        obuf[...] = h2
        pltpu.make_async_copy(obuf, out_hbm.at[t], osem).start()

        @pl.when(t + 2 < S)
        def _():
            pltpu.make_async_copy(z_hbm.at[t + 2], zbuf, zsem).start()

    def body(idx, carry):
        t0 = idx * 2
        one_step(t0, zb0, zs0, ob0, os0, ga_ref, gb_ref)
        one_step(t0 + 1, zb1, zs1, ob1, os1, gb_ref, ga_ref)
        return carry

    jax.lax.fori_loop(0, S // 2, body, 0)

    pltpu.make_async_copy(ob0, out_hbm.at[S - 2], os0).wait()
    pltpu.make_async_copy(ob1, out_hbm.at[S - 1], os1).wait()
    hT_ref[...] = h_ref[...]
    cT_ref[...] = c_ref[...]


def kernel(input, input_embed, W_ih_0, W_hh_0, b_ih_0, b_hh_0,
           W_ih_1, W_hh_1, b_ih_1, b_hh_1):
    tok_sb = jnp.swapaxes(input, 0, 1)
    m_sb = (tok_sb < SPLIT).astype(jnp.float32)
    x_sb = jnp.swapaxes(input_embed, 0, 1).reshape(S * B, E)
    b0 = (b_ih_0 + b_hh_0).reshape(1, G4)
    b1 = (b_ih_1 + b_hh_1).reshape(1, G4)
    BN = 512
    NB = G4 // BN
    zsel = pl.pallas_call(
        _proj_kernel,
        grid=(NB,),
        in_specs=[
            pl.BlockSpec((S * B, E), lambda n: (0, 0)),
            pl.BlockSpec((BN, E), lambda n: (n, 0)),
            pl.BlockSpec((BN, E), lambda n: (n, 0)),
            pl.BlockSpec((1, BN), lambda n: (0, n)),
            pl.BlockSpec((1, BN), lambda n: (0, n)),
            pl.BlockSpec((S * B, 1), lambda n: (0, 0)),
        ],
        out_specs=pl.BlockSpec((S * B, BN), lambda n: (0, n)),
        out_shape=jax.ShapeDtypeStruct((S * B, G4), jnp.bfloat16),
    )(x_sb, W_ih_0, W_ih_1, b0, b1, m_sb.reshape(S * B, 1))

    z3 = zsel.reshape(S, B, G4)

    out_sbh, hT, cT = pl.pallas_call(
        _step_kernel,
        in_specs=[
            pl.BlockSpec(memory_space=pl.ANY),
            pl.BlockSpec((S + 1, B, 1), lambda: (0, 0, 0)),
            pl.BlockSpec((G4, H), lambda: (0, 0)),
            pl.BlockSpec((G4, H), lambda: (0, 0)),
        ],
        out_specs=[
            pl.BlockSpec(memory_space=pl.ANY),
            pl.BlockSpec((B, H), lambda: (0, 0)),
            pl.BlockSpec((B, H), lambda: (0, 0)),
        ],
        out_shape=[
            jax.ShapeDtypeStruct((S, B, H), jnp.float32),
            jax.ShapeDtypeStruct((B, H), jnp.float32),
            jax.ShapeDtypeStruct((B, H), jnp.float32),
        ],
        scratch_shapes=[
            pltpu.VMEM((2 * H, G4), jnp.bfloat16),
            pltpu.VMEM((B, H), jnp.float32),
            pltpu.VMEM((B, H), jnp.float32),
            pltpu.VMEM((B, G4), jnp.float32),
            pltpu.VMEM((B, G4), jnp.float32),
            pltpu.VMEM((B, G4), jnp.bfloat16),
            pltpu.VMEM((B, G4), jnp.bfloat16),
            pltpu.VMEM((B, H), jnp.float32),
            pltpu.VMEM((B, H), jnp.float32),
            pltpu.SemaphoreType.DMA,
            pltpu.SemaphoreType.DMA,
            pltpu.SemaphoreType.DMA,
            pltpu.SemaphoreType.DMA,
        ],
    )(z3, jnp.concatenate([m_sb, jnp.zeros((1, B), jnp.float32)],
                          axis=0).reshape(S + 1, B, 1),
      W_hh_0, W_hh_1)

    return jnp.swapaxes(out_sbh, 0, 1), hT, cT


# speedup vs baseline: 1.0399x; 1.0399x over previous
import jax
import jax.numpy as jnp
from jax.experimental import pallas as pl
from jax.experimental.pallas import tpu as pltpu

B, S, E, H = 64, 32, 1024, 1024
G4 = 4 * H
SPLIT = 50

_DN = (((1,), (0,)), ((), ()))
_DNT = (((1,), (1,)), ((), ()))


def _proj_kernel(x_ref, w0_ref, w1_ref, wh0_ref, wh1_ref, b0_ref, b1_ref,
                 m_ref, out_ref, wbf_ref):
    x = x_ref[...]
    z0 = jax.lax.dot_general(x, w0_ref[...], _DNT,
                             preferred_element_type=jnp.float32)
    z1 = jax.lax.dot_general(x, w1_ref[...], _DNT,
                             preferred_element_type=jnp.float32)
    m = m_ref[...]
    out_ref[...] = (m * (z0 + b0_ref[...])
                    + (1.0 - m) * (z1 + b1_ref[...])).astype(jnp.bfloat16)
    wbf_ref[...] = jnp.concatenate(
        [jnp.swapaxes(wh0_ref[...], 0, 1),
         jnp.swapaxes(wh1_ref[...], 0, 1)], axis=0).astype(jnp.bfloat16)


def _step_kernel(z_hbm, m_ref, wbf_ref, out_hbm, hT_ref, cT_ref,
                 h_ref, c_ref, zb_ref, ob0, ob1,
                 zsem, os0, os1):
    h_ref[...] = jnp.zeros_like(h_ref)
    c_ref[...] = jnp.zeros_like(c_ref)

    def prefetch(t, carry):
        pltpu.make_async_copy(z_hbm.at[t], zb_ref.at[t], zsem).start()
        return carry

    jax.lax.fori_loop(0, S, prefetch, 0)

    def one_step(t, obuf, osem):
        pltpu.make_async_copy(z_hbm.at[t], zb_ref.at[t], zsem).wait()
        h = h_ref[...]
        m = m_ref[t]
        hcat = jnp.concatenate([m * h, h - m * h], axis=1).astype(jnp.bfloat16)
        g = jax.lax.dot_general(hcat, wbf_ref[...], _DN,
                                preferred_element_type=jnp.float32)
        gates = g + zb_ref[t]
        i = jax.nn.sigmoid(gates[:, :H])
        f = jax.nn.sigmoid(gates[:, H:2 * H])
        gg = jnp.tanh(gates[:, 2 * H:3 * H])
        o = jax.nn.sigmoid(gates[:, 3 * H:])
        c = f * c_ref[...] + i * gg
        h2 = o * jnp.tanh(c)
        c_ref[...] = c
        h_ref[...] = h2

        @pl.when(t >= 2)
        def _():
            pltpu.make_async_copy(obuf, out_hbm.at[t], osem).wait()

        obuf[...] = h2
        pltpu.make_async_copy(obuf, out_hbm.at[t], osem).start()

    def body(idx, carry):
        t0 = idx * 2
        one_step(t0, ob0, os0)
        one_step(t0 + 1, ob1, os1)
        return carry

    jax.lax.fori_loop(0, S // 2, body, 0)

    pltpu.make_async_copy(ob0, out_hbm.at[S - 2], os0).wait()
    pltpu.make_async_copy(ob1, out_hbm.at[S - 1], os1).wait()
    hT_ref[...] = h_ref[...]
    cT_ref[...] = c_ref[...]


def kernel(input, input_embed, W_ih_0, W_hh_0, b_ih_0, b_hh_0,
           W_ih_1, W_hh_1, b_ih_1, b_hh_1):
    tok_sb = jnp.swapaxes(input, 0, 1)
    m_sb = (tok_sb < SPLIT).astype(jnp.float32)
    x_sb = jnp.swapaxes(input_embed, 0, 1).reshape(S * B, E)
    b0 = (b_ih_0 + b_hh_0).reshape(1, G4)
    b1 = (b_ih_1 + b_hh_1).reshape(1, G4)
    BN = 512
    NB = G4 // BN
    zsel, wbf = pl.pallas_call(
        _proj_kernel,
        grid=(NB,),
        in_specs=[
            pl.BlockSpec((S * B, E), lambda n: (0, 0)),
            pl.BlockSpec((BN, E), lambda n: (n, 0)),
            pl.BlockSpec((BN, E), lambda n: (n, 0)),
            pl.BlockSpec((BN, H), lambda n: (n, 0)),
            pl.BlockSpec((BN, H), lambda n: (n, 0)),
            pl.BlockSpec((1, BN), lambda n: (0, n)),
            pl.BlockSpec((1, BN), lambda n: (0, n)),
            pl.BlockSpec((S * B, 1), lambda n: (0, 0)),
        ],
        out_specs=[
            pl.BlockSpec((S * B, BN), lambda n: (0, n)),
            pl.BlockSpec((2 * H, BN), lambda n: (0, n)),
        ],
        out_shape=[
            jax.ShapeDtypeStruct((S * B, G4), jnp.bfloat16),
            jax.ShapeDtypeStruct((2 * H, G4), jnp.bfloat16),
        ],
    )(x_sb, W_ih_0, W_ih_1, W_hh_0, W_hh_1, b0, b1, m_sb.reshape(S * B, 1))

    z3 = zsel.reshape(S, B, G4)

    out_sbh, hT, cT = pl.pallas_call(
        _step_kernel,
        in_specs=[
            pl.BlockSpec(memory_space=pl.ANY),
            pl.BlockSpec((S, B, 1), lambda: (0, 0, 0)),
            pl.BlockSpec((2 * H, G4), lambda: (0, 0)),
        ],
        out_specs=[
            pl.BlockSpec(memory_space=pl.ANY),
            pl.BlockSpec((B, H), lambda: (0, 0)),
            pl.BlockSpec((B, H), lambda: (0, 0)),
        ],
        out_shape=[
            jax.ShapeDtypeStruct((S, B, H), jnp.float32),
            jax.ShapeDtypeStruct((B, H), jnp.float32),
            jax.ShapeDtypeStruct((B, H), jnp.float32),
        ],
        scratch_shapes=[
            pltpu.VMEM((B, H), jnp.float32),
            pltpu.VMEM((B, H), jnp.float32),
            pltpu.VMEM((S, B, G4), jnp.bfloat16),
            pltpu.VMEM((B, H), jnp.float32),
            pltpu.VMEM((B, H), jnp.float32),
            pltpu.SemaphoreType.DMA,
            pltpu.SemaphoreType.DMA,
            pltpu.SemaphoreType.DMA,
        ],
    )(z3, m_sb.reshape(S, B, 1), wbf)

    return jnp.swapaxes(out_sbh, 0, 1), hT, cT
